# baseline (speedup 1.0000x reference)
import functools

import jax
import jax.numpy as jnp
from jax import lax
from jax.experimental import pallas as pl
from jax.experimental.pallas import tpu as pltpu
from jax.experimental.pallas import tpu_sc as plsc

ANGSTROM_TO_BOHR = 1.8897261258369282
CUTOFF = 5.2

NC = 2
NS = 16
L = 16
NW = NC * NS


def _f32(x):
    return jnp.float32(x)


@functools.partial(jax.jit, static_argnames=("n_mol", "num_atoms"))
def _sc_repulsion(packed, i0, i1, dist, ya16, nsa16, kr16, *, n_mol, num_atoms):
    E = i0.shape[0]
    per_w = E // NW
    assert per_w * NW == E
    chunk = 10000 if per_w % 20000 == 0 else per_w
    n_pairs = per_w // (2 * chunk)
    n_vec = chunk // L
    assert chunk % L == 0 and per_w % (2 * chunk) == 0

    W = -(-n_mol // (NS * L)) * (NS * L)
    cols_per_tile = W // NS
    n_packed = packed.shape[0]

    rc = CUTOFF * ANGSTROM_TO_BOHR
    inv_rc2 = 1.0 / (rc * rc)
    inv_na = 1.0 / num_atoms

    mesh = plsc.VectorSubcoreMesh(core_axis_name="c", subcore_axis_name="s")

    @functools.partial(
        pl.kernel,
        out_type=jax.ShapeDtypeStruct((NC, W), jnp.float32),
        mesh=mesh,
        compiler_params=pltpu.CompilerParams(needs_layout_passes=False),
        scratch_types=[
            pltpu.VMEM((n_packed,), jnp.int32),
            pltpu.VMEM((16,), jnp.float32),
            pltpu.VMEM((16,), jnp.float32),
            pltpu.VMEM((16,), jnp.float32),
            [pltpu.VMEM((chunk,), jnp.int32) for _ in range(2)],
            [pltpu.VMEM((chunk,), jnp.int32) for _ in range(2)],
            [pltpu.VMEM((chunk,), jnp.float32) for _ in range(2)],
            [pltpu.SemaphoreType.DMA for _ in range(2)],
            pltpu.VMEM((NS * W,), jnp.float32),
            pltpu.VMEM((W,), jnp.float32),
            pltpu.VMEM_SHARED((NS, W), jnp.float32),
            pltpu.VMEM((NS, cols_per_tile), jnp.float32),
            pltpu.VMEM((cols_per_tile,), jnp.float32),
        ],
    )
    def body(packed_hbm, i0_hbm, i1_hbm, d_hbm, ya_hbm, nsa_hbm, kr_hbm,
             out_hbm, packed_v, ya_v, nsa_v, kr_v, i0_v, i1_v, d_v, sems,
             accb, red_v, shared, tmp_v, outr_v):
        cid = lax.axis_index("c")
        sid = lax.axis_index("s")
        wid = sid * NC + cid
        base = wid * per_w

        def start(slot, off):
            pltpu.async_copy(i0_hbm.at[pl.ds(off, chunk)], i0_v[slot], sems[slot])
            pltpu.async_copy(i1_hbm.at[pl.ds(off, chunk)], i1_v[slot], sems[slot])
            pltpu.async_copy(d_hbm.at[pl.ds(off, chunk)], d_v[slot], sems[slot])

        def drain(slot):
            pltpu.make_async_copy(i0_hbm.at[pl.ds(0, chunk)], i0_v[slot], sems[slot]).wait()
            pltpu.make_async_copy(i1_hbm.at[pl.ds(0, chunk)], i1_v[slot], sems[slot]).wait()
            pltpu.make_async_copy(d_hbm.at[pl.ds(0, chunk)], d_v[slot], sems[slot]).wait()

        start(0, base)
        pltpu.sync_copy(packed_hbm, packed_v)
        pltpu.sync_copy(ya_hbm, ya_v)
        pltpu.sync_copy(nsa_hbm, nsa_v)
        pltpu.sync_copy(kr_hbm, kr_v)

        zeros = jnp.zeros((L,), jnp.float32)
        lane = lax.broadcasted_iota(jnp.int32, (L,), 0)
        lane_w = lane * W

        def zbody(j, _):
            accb[pl.ds(j * L, L)] = zeros
            return 0

        lax.fori_loop(0, NS * W // L, zbody, 0)

        def process(slot):
            i0b, i1b, db = i0_v[slot], i1_v[slot], d_v[slot]

            @plsc.parallel_loop(0, n_vec, unroll=6)
            def vbody(i):
                sl = pl.ds(i * L, L)
                a0 = i0b[sl]
                a1 = i1b[sl]
                w0 = plsc.load_gather(packed_v, [a0 >> 4])
                w1 = plsc.load_gather(packed_v, [a1 >> 4])
                s0 = (w0 >> ((a0 & 15) << 1)) & 3
                s1 = (w1 >> ((a1 & 15) << 1)) & 3
                c = (s0 << 2) + s1
                ya = plsc.load_gather(ya_v, [c])
                nsa = plsc.load_gather(nsa_v, [c])
                kr = plsc.load_gather(kr_v, [c])

                d = db[sl] * _f32(ANGSTROM_TO_BOHR)
                bi = plsc.bitcast(d, jnp.int32)
                r = plsc.bitcast(jnp.int32(0x5F3759DF) - (bi >> 1), jnp.float32)
                r = r * (_f32(1.5) - _f32(0.5) * d * r * r)
                r = r * (_f32(1.5) - _f32(0.5) * d * r * r)
                d2 = d * d
                inv_d = r * r
                dp = jnp.where(kr < _f32(1.25), d, d2 * r)
                x2 = jnp.minimum(d2 * _f32(inv_rc2), _f32(1.0 - 1e-6))
                arg = nsa * dp + (_f32(1.0) - _f32(1.0) / (_f32(1.0) - x2))
                en = ya * inv_d * jnp.exp(arg)

                q = (a0.astype(jnp.float32) * _f32(inv_na) + _f32(0.5 * inv_na)
                     ).astype(jnp.int32)
                plsc.addupdate_scatter(accb, [lane_w + q], en)

        def pair_body(p, _):
            off = base + 2 * p * chunk
            start(1, off + chunk)
            drain(0)
            process(0)

            @pl.when(p < n_pairs - 1)
            def _():
                start(0, off + 2 * chunk)

            drain(1)
            process(1)
            return 0

        lax.fori_loop(0, n_pairs, pair_body, 0)

        def rbody(j, _):
            sl = j * L
            s = accb[pl.ds(sl, L)]
            for b in range(1, NS):
                s = s + accb[pl.ds(b * W + sl, L)]
            red_v[pl.ds(sl, L)] = s
            return 0

        lax.fori_loop(0, W // L, rbody, 0)
        pltpu.sync_copy(red_v, shared.at[sid])
        plsc.subcore_barrier()

        col0 = sid * cols_per_tile
        pltpu.sync_copy(shared.at[:, pl.ds(col0, cols_per_tile)], tmp_v)

        def r2body(j, _):
            sl = pl.ds(j * L, L)
            s = tmp_v[0, sl]
            for b in range(1, NS):
                s = s + tmp_v[b, sl]
            outr_v[sl] = s
            return 0

        lax.fori_loop(0, cols_per_tile // L, r2body, 0)
        pltpu.sync_copy(outr_v, out_hbm.at[cid, pl.ds(col0, cols_per_tile)])

    return body(packed, i0, i1, dist, ya16, nsa16, kr16)


def kernel(species, atom_index12, distances, y_ab, sqrt_alpha_ab, k_rep_ab):
    n_mol, num_atoms = species.shape
    fs = species.reshape(-1).astype(jnp.uint32)
    n_flat = fs.shape[0]
    assert n_flat % L == 0
    shifts = jnp.arange(L, dtype=jnp.uint32) * 2
    packed = (fs.reshape(-1, L) << shifts).sum(axis=1, dtype=jnp.uint32)
    n_packed = -(-packed.shape[0] // 16) * 16
    packed = jnp.zeros((n_packed,), jnp.uint32).at[: packed.shape[0]].set(packed)
    packed = packed.astype(jnp.int32)

    i0 = atom_index12[0].astype(jnp.int32)
    i1 = atom_index12[1].astype(jnp.int32)
    out2 = _sc_repulsion(
        packed, i0, i1, distances.astype(jnp.float32),
        y_ab.reshape(-1).astype(jnp.float32),
        (-sqrt_alpha_ab.reshape(-1)).astype(jnp.float32),
        k_rep_ab.reshape(-1).astype(jnp.float32),
        n_mol=n_mol, num_atoms=num_atoms,
    )
    return out2[:, :n_mol].sum(axis=0)


# speedup vs baseline: 1.0211x; 1.0211x over previous
import functools

import jax
import jax.numpy as jnp
from jax import lax
from jax.experimental import pallas as pl
from jax.experimental.pallas import tpu as pltpu
from jax.experimental.pallas import tpu_sc as plsc

ANGSTROM_TO_BOHR = 1.8897261258369282
CUTOFF = 5.2

NC = 2
NS = 16
L = 16
NW = NC * NS


def _f32(x):
    return jnp.float32(x)


@functools.partial(jax.jit, static_argnames=("n_mol", "num_atoms"))
def _sc_repulsion(packed, i0, i1, dist, ya16, nsa16, kr16, *, n_mol, num_atoms):
    E = i0.shape[0]
    per_w = E // NW
    assert per_w * NW == E
    chunk = 10000 if per_w % 20000 == 0 else per_w
    n_pairs = per_w // (2 * chunk)
    n_vec = chunk // L
    assert chunk % L == 0 and per_w % (2 * chunk) == 0

    W = -(-n_mol // (NS * L)) * (NS * L)
    cols_per_tile = W // NS
    n_packed = packed.shape[0]

    rc = CUTOFF * ANGSTROM_TO_BOHR
    inv_rc2 = 1.0 / (rc * rc)
    inv_na = 1.0 / num_atoms

    mesh = plsc.VectorSubcoreMesh(core_axis_name="c", subcore_axis_name="s")

    @functools.partial(
        pl.kernel,
        out_type=jax.ShapeDtypeStruct((NC, W), jnp.float32),
        mesh=mesh,
        compiler_params=pltpu.CompilerParams(needs_layout_passes=False),
        scratch_types=[
            pltpu.VMEM((n_packed,), jnp.int32),
            pltpu.VMEM((16,), jnp.float32),
            pltpu.VMEM((16,), jnp.float32),
            pltpu.VMEM((16,), jnp.float32),
            [pltpu.VMEM((chunk,), jnp.int32) for _ in range(2)],
            [pltpu.VMEM((chunk,), jnp.int32) for _ in range(2)],
            [pltpu.VMEM((chunk,), jnp.float32) for _ in range(2)],
            [pltpu.SemaphoreType.DMA for _ in range(2)],
            pltpu.VMEM((NS * W,), jnp.float32),
            pltpu.VMEM((W,), jnp.float32),
            pltpu.VMEM_SHARED((NS, W), jnp.float32),
            pltpu.VMEM((NS, cols_per_tile), jnp.float32),
            pltpu.VMEM((cols_per_tile,), jnp.float32),
        ],
    )
    def body(packed_hbm, i0_hbm, i1_hbm, d_hbm, ya_hbm, nsa_hbm, kr_hbm,
             out_hbm, packed_v, ya_v, nsa_v, kr_v, i0_v, i1_v, d_v, sems,
             accb, red_v, shared, tmp_v, outr_v):
        cid = lax.axis_index("c")
        sid = lax.axis_index("s")
        wid = sid * NC + cid
        base = wid * per_w

        def start(slot, off):
            pltpu.async_copy(i0_hbm.at[pl.ds(off, chunk)], i0_v[slot], sems[slot])
            pltpu.async_copy(i1_hbm.at[pl.ds(off, chunk)], i1_v[slot], sems[slot])
            pltpu.async_copy(d_hbm.at[pl.ds(off, chunk)], d_v[slot], sems[slot])

        def drain(slot):
            pltpu.make_async_copy(i0_hbm.at[pl.ds(0, chunk)], i0_v[slot], sems[slot]).wait()
            pltpu.make_async_copy(i1_hbm.at[pl.ds(0, chunk)], i1_v[slot], sems[slot]).wait()
            pltpu.make_async_copy(d_hbm.at[pl.ds(0, chunk)], d_v[slot], sems[slot]).wait()

        start(0, base)
        pltpu.sync_copy(packed_hbm, packed_v)
        pltpu.sync_copy(ya_hbm, ya_v)
        pltpu.sync_copy(nsa_hbm, nsa_v)
        pltpu.sync_copy(kr_hbm, kr_v)

        zeros = jnp.zeros((L,), jnp.float32)
        lane = lax.broadcasted_iota(jnp.int32, (L,), 0)
        lane_w = lane * W

        def zbody(j, _):
            accb[pl.ds(j * L, L)] = zeros
            return 0

        lax.fori_loop(0, NS * W // L, zbody, 0)

        def process(slot):
            i0b, i1b, db = i0_v[slot], i1_v[slot], d_v[slot]

            @plsc.parallel_loop(0, n_vec, unroll=4)
            def vbody(i):
                sl = pl.ds(i * L, L)
                a0 = i0b[sl]
                a1 = i1b[sl]
                w0 = plsc.load_gather(packed_v, [a0 >> 4])
                w1 = plsc.load_gather(packed_v, [a1 >> 4])
                s0 = (w0 >> ((a0 & 15) << 1)) & 3
                s1 = (w1 >> ((a1 & 15) << 1)) & 3
                c = (s0 << 2) + s1
                ya = plsc.load_gather(ya_v, [c])
                nsa = plsc.load_gather(nsa_v, [c])

                d = db[sl] * _f32(ANGSTROM_TO_BOHR)
                bi = plsc.bitcast(d, jnp.int32)
                r = plsc.bitcast(jnp.int32(0x5F3759DF) - (bi >> 1), jnp.float32)
                r = r * (_f32(1.5) - _f32(0.5) * d * r * r)
                r = r * (_f32(1.5) - _f32(0.5) * d * r * r)
                d2 = d * d
                inv_d = r * r
                dp = jnp.where(c == 0, d, d2 * r)
                x2 = jnp.minimum(d2 * _f32(inv_rc2), _f32(1.0 - 1e-6))
                arg = nsa * dp + (_f32(1.0) - _f32(1.0) / (_f32(1.0) - x2))
                en = ya * inv_d * jnp.exp(arg)

                q = (a0.astype(jnp.float32) * _f32(inv_na) + _f32(0.5 * inv_na)
                     ).astype(jnp.int32)
                plsc.addupdate_scatter(accb, [lane_w + q], en)

        def pair_body(p, _):
            off = base + 2 * p * chunk
            start(1, off + chunk)
            drain(0)
            process(0)

            @pl.when(p < n_pairs - 1)
            def _():
                start(0, off + 2 * chunk)

            drain(1)
            process(1)
            return 0

        lax.fori_loop(0, n_pairs, pair_body, 0)

        def rbody(j, _):
            sl = j * L
            s = accb[pl.ds(sl, L)]
            for b in range(1, NS):
                s = s + accb[pl.ds(b * W + sl, L)]
            red_v[pl.ds(sl, L)] = s
            return 0

        lax.fori_loop(0, W // L, rbody, 0)
        pltpu.sync_copy(red_v, shared.at[sid])
        plsc.subcore_barrier()

        col0 = sid * cols_per_tile
        pltpu.sync_copy(shared.at[:, pl.ds(col0, cols_per_tile)], tmp_v)

        def r2body(j, _):
            sl = pl.ds(j * L, L)
            s = tmp_v[0, sl]
            for b in range(1, NS):
                s = s + tmp_v[b, sl]
            outr_v[sl] = s
            return 0

        lax.fori_loop(0, cols_per_tile // L, r2body, 0)
        pltpu.sync_copy(outr_v, out_hbm.at[cid, pl.ds(col0, cols_per_tile)])

    return body(packed, i0, i1, dist, ya16, nsa16, kr16)


def kernel(species, atom_index12, distances, y_ab, sqrt_alpha_ab, k_rep_ab):
    n_mol, num_atoms = species.shape
    fs = species.reshape(-1).astype(jnp.uint32)
    n_flat = fs.shape[0]
    assert n_flat % L == 0
    shifts = jnp.arange(L, dtype=jnp.uint32) * 2
    packed = (fs.reshape(-1, L) << shifts).sum(axis=1, dtype=jnp.uint32)
    n_packed = -(-packed.shape[0] // 16) * 16
    packed = jnp.zeros((n_packed,), jnp.uint32).at[: packed.shape[0]].set(packed)
    packed = packed.astype(jnp.int32)

    i0 = atom_index12[0].astype(jnp.int32)
    i1 = atom_index12[1].astype(jnp.int32)
    out2 = _sc_repulsion(
        packed, i0, i1, distances.astype(jnp.float32),
        y_ab.reshape(-1).astype(jnp.float32),
        (-sqrt_alpha_ab.reshape(-1)).astype(jnp.float32),
        k_rep_ab.reshape(-1).astype(jnp.float32),
        n_mol=n_mol, num_atoms=num_atoms,
    )
    return out2[:, :n_mol].sum(axis=0)


# speedup vs baseline: 1.0297x; 1.0085x over previous
import functools

import jax
import jax.numpy as jnp
from jax import lax
from jax.experimental import pallas as pl
from jax.experimental.pallas import tpu as pltpu
from jax.experimental.pallas import tpu_sc as plsc

ANGSTROM_TO_BOHR = 1.8897261258369282
CUTOFF = 5.2

NC = 2
NS = 16
L = 16
NW = NC * NS


def _f32(x):
    return jnp.float32(x)


@functools.partial(jax.jit, static_argnames=("n_mol", "num_atoms"))
def _sc_repulsion(packed, i0, i1, dist, ya16, nsa16, *, n_mol, num_atoms):
    E = i0.shape[0]
    per_w = E // NW
    assert per_w * NW == E
    chunk = 10000 if per_w % 20000 == 0 else per_w
    n_pairs = per_w // (2 * chunk)
    n_vec = chunk // L
    assert chunk % L == 0 and per_w % (2 * chunk) == 0

    W = -(-n_mol // (NS * L)) * (NS * L)
    cols_per_tile = W // NS
    n_packed = packed.shape[0]

    rc = CUTOFF * ANGSTROM_TO_BOHR
    inv_rc2 = 1.0 / (rc * rc)
    inv_na = 1.0 / num_atoms

    mesh = plsc.VectorSubcoreMesh(core_axis_name="c", subcore_axis_name="s")

    @functools.partial(
        pl.kernel,
        out_type=jax.ShapeDtypeStruct((NC, W), jnp.float32),
        mesh=mesh,
        compiler_params=pltpu.CompilerParams(needs_layout_passes=False),
        scratch_types=[
            pltpu.VMEM((n_packed,), jnp.int32),
            pltpu.VMEM((16,), jnp.float32),
            pltpu.VMEM((16,), jnp.float32),
            [pltpu.VMEM((chunk,), jnp.int32) for _ in range(2)],
            [pltpu.VMEM((chunk,), jnp.int32) for _ in range(2)],
            [pltpu.VMEM((chunk,), jnp.float32) for _ in range(2)],
            [pltpu.SemaphoreType.DMA for _ in range(2)],
            pltpu.VMEM((NS * W,), jnp.float32),
            pltpu.VMEM((W,), jnp.float32),
            pltpu.VMEM_SHARED((NS, W), jnp.float32),
            pltpu.VMEM((NS, cols_per_tile), jnp.float32),
            pltpu.VMEM((cols_per_tile,), jnp.float32),
        ],
    )
    def body(packed_hbm, i0_hbm, i1_hbm, d_hbm, ya_hbm, nsa_hbm,
             out_hbm, packed_v, ya_v, nsa_v, i0_v, i1_v, d_v, sems,
             accb, red_v, shared, tmp_v, outr_v):
        cid = lax.axis_index("c")
        sid = lax.axis_index("s")
        wid = sid * NC + cid
        base = wid * per_w

        def start(slot, off):
            pltpu.async_copy(i0_hbm.at[pl.ds(off, chunk)], i0_v[slot], sems[slot])
            pltpu.async_copy(i1_hbm.at[pl.ds(off, chunk)], i1_v[slot], sems[slot])
            pltpu.async_copy(d_hbm.at[pl.ds(off, chunk)], d_v[slot], sems[slot])

        def drain(slot):
            pltpu.make_async_copy(i0_hbm.at[pl.ds(0, chunk)], i0_v[slot], sems[slot]).wait()
            pltpu.make_async_copy(i1_hbm.at[pl.ds(0, chunk)], i1_v[slot], sems[slot]).wait()
            pltpu.make_async_copy(d_hbm.at[pl.ds(0, chunk)], d_v[slot], sems[slot]).wait()

        start(0, base)
        pltpu.sync_copy(packed_hbm, packed_v)
        pltpu.sync_copy(ya_hbm, ya_v)
        pltpu.sync_copy(nsa_hbm, nsa_v)

        zeros = jnp.zeros((L,), jnp.float32)
        lane = lax.broadcasted_iota(jnp.int32, (L,), 0)
        lane_w = lane * W

        def zbody(j, _):
            accb[pl.ds(j * L, L)] = zeros
            return 0

        lax.fori_loop(0, NS * W // L, zbody, 0)

        def process(slot):
            i0b, i1b, db = i0_v[slot], i1_v[slot], d_v[slot]

            @plsc.parallel_loop(0, n_vec, unroll=4)
            def vbody(i):
                sl = pl.ds(i * L, L)
                a0 = i0b[sl]
                a1 = i1b[sl]
                w0 = plsc.load_gather(packed_v, [a0 >> 4])
                w1 = plsc.load_gather(packed_v, [a1 >> 4])
                s0 = (w0 >> ((a0 & 15) << 1)) & 3
                s1 = (w1 >> ((a1 & 15) << 1)) & 3
                c = (s0 << 2) + s1
                ya = plsc.load_gather(ya_v, [c])
                nsa = plsc.load_gather(nsa_v, [c])

                d = db[sl] * _f32(ANGSTROM_TO_BOHR)
                bi = plsc.bitcast(d, jnp.int32)
                r = plsc.bitcast(jnp.int32(0x5F3759DF) - (bi >> 1), jnp.float32)
                r = r * (_f32(1.5) - _f32(0.5) * d * r * r)
                r = r * (_f32(1.5) - _f32(0.5) * d * r * r)
                d2 = d * d
                inv_d = r * r
                dp = jnp.where(c == 0, d, d2 * r)
                x2 = jnp.minimum(d2 * _f32(inv_rc2), _f32(1.0 - 1e-6))
                arg = nsa * dp + (_f32(1.0) - _f32(1.0) / (_f32(1.0) - x2))
                en = ya * inv_d * jnp.exp(arg)

                q = (a0.astype(jnp.float32) * _f32(inv_na) + _f32(0.5 * inv_na)
                     ).astype(jnp.int32)
                plsc.addupdate_scatter(accb, [lane_w + q], en)

        def pair_body(p, _):
            off = base + 2 * p * chunk
            start(1, off + chunk)
            drain(0)
            process(0)

            @pl.when(p < n_pairs - 1)
            def _():
                start(0, off + 2 * chunk)

            drain(1)
            process(1)
            return 0

        lax.fori_loop(0, n_pairs, pair_body, 0)

        def rbody(j, _):
            sl = j * L
            s = accb[pl.ds(sl, L)]
            for b in range(1, NS):
                s = s + accb[pl.ds(b * W + sl, L)]
            red_v[pl.ds(sl, L)] = s
            return 0

        lax.fori_loop(0, W // L, rbody, 0)
        pltpu.sync_copy(red_v, shared.at[sid])
        plsc.subcore_barrier()

        col0 = sid * cols_per_tile
        pltpu.sync_copy(shared.at[:, pl.ds(col0, cols_per_tile)], tmp_v)

        def r2body(j, _):
            sl = pl.ds(j * L, L)
            s = tmp_v[0, sl]
            for b in range(1, NS):
                s = s + tmp_v[b, sl]
            outr_v[sl] = s
            return 0

        lax.fori_loop(0, cols_per_tile // L, r2body, 0)
        pltpu.sync_copy(outr_v, out_hbm.at[cid, pl.ds(col0, cols_per_tile)])

    return body(packed, i0, i1, dist, ya16, nsa16)


def kernel(species, atom_index12, distances, y_ab, sqrt_alpha_ab, k_rep_ab):
    n_mol, num_atoms = species.shape
    fs = species.reshape(-1).astype(jnp.uint32)
    n_flat = fs.shape[0]
    assert n_flat % L == 0
    shifts = jnp.arange(L, dtype=jnp.uint32) * 2
    packed = (fs.reshape(-1, L) << shifts).sum(axis=1, dtype=jnp.uint32)
    n_packed = -(-packed.shape[0] // 16) * 16
    packed = jnp.zeros((n_packed,), jnp.uint32).at[: packed.shape[0]].set(packed)
    packed = packed.astype(jnp.int32)

    i0 = atom_index12[0].astype(jnp.int32)
    i1 = atom_index12[1].astype(jnp.int32)
    out2 = _sc_repulsion(
        packed, i0, i1, distances.astype(jnp.float32),
        y_ab.reshape(-1).astype(jnp.float32),
        (-sqrt_alpha_ab.reshape(-1)).astype(jnp.float32),
        n_mol=n_mol, num_atoms=num_atoms,
    )
    return out2[:, :n_mol].sum(axis=0)
